# baseline (device time: 100357 ns/iter reference)
import jax
import jax.numpy as jnp
from jax import lax
from jax.experimental import pallas as pl
from jax.experimental.pallas import tpu as pltpu

N_DEV = 4
PH = 4


def kernel(x, w_mat):
    M, _ = x.shape
    _, N = w_mat.shape
    Mo = M // N_DEV
    Npc = N // (2 * PH)

    def body(x_ref, w_ref, out_ref,
             sbufA, rbuf1A, rbuf2A, rbuf2bA, sbufB, rbuf1B, rbuf2B, rbuf2bB,
             amax_out, amax_in,
             s1_send_sems, s1_recv_sems, s2_send_sems, s2_recv_sems,
             credit1, credit2,
             amax_send_sems, amax_recv_sems):
        my = lax.axis_index("i")
        left = lax.rem(my + 3, N_DEV)
        right = lax.rem(my + 1, N_DEV)
        barrier = pltpu.get_barrier_semaphore()

        pA, pB = my ^ 1, 3 - my
        schemes = (
            dict(idx=0, p1=pA, p2=3 - my, s1_chunks=(3 - pA, pA),
                 sbuf=sbufA, rbuf1=rbuf1A, rbuf2=rbuf2A, rbuf2b=rbuf2bA,
                 col0=0),
            dict(idx=1, p1=pB, p2=my ^ 1, s1_chunks=(pB ^ 1, pB),
                 sbuf=sbufB, rbuf1=rbuf1B, rbuf2=rbuf2B, rbuf2b=rbuf2bB,
                 col0=N // 2),
        )

        def colsl(s, p):
            return pl.ds(s["col0"] + p * Npc, Npc)

        def chunk_dot(c, col):
            return jnp.dot(x_ref[pl.ds(c * Mo, Mo), :], w_ref[:, col],
                           preferred_element_type=jnp.float32)

        for slot in (0, 1):
            for s in schemes:
                s["sbuf"][slot] = chunk_dot(
                    s["s1_chunks"][slot], colsl(s, 0)).astype(jnp.bfloat16)

        for nbr in (left, right):
            pl.semaphore_signal(
                barrier, inc=1, device_id=(nbr,),
                device_id_type=pl.DeviceIdType.MESH)
        pl.semaphore_wait(barrier, 2)

        s2_prev = [None, None]
        for p in range(PH):
            base = 2 * (p % 2)
            for s in schemes:
                if p > 0:
                    pl.semaphore_wait(credit1.at[s["idx"]], 1)
            s1 = {0: [], 1: []}
            for slot in (0, 1):
                for s in schemes:
                    rdma = pltpu.make_async_remote_copy(
                        src_ref=s["sbuf"].at[base + slot],
                        dst_ref=s["rbuf1"].at[slot],
                        send_sem=s1_send_sems.at[p, s["idx"], slot],
                        recv_sem=s1_recv_sems.at[p, s["idx"], slot],
                        device_id=(s["p1"],),
                        device_id_type=pl.DeviceIdType.MESH,
                    )
                    rdma.start()
                    s1[s["idx"]].append(rdma)

            for s in schemes:
                out_ref[:, colsl(s, p)] = chunk_dot(my, colsl(s, p))
            for s in schemes:
                if p > 0:
                    s2_prev[s["idx"]].wait()
                    pcol = colsl(s, p - 1)
                    out_ref[:, pcol] = jnp.maximum(
                        out_ref[:, pcol]
                        + s["rbuf2b"][...].astype(jnp.float32), 0.0)
                s["rbuf2"][...] = chunk_dot(s["p2"], colsl(s, p))

            for s in schemes:
                s1[s["idx"]][0].wait()
                s["sbuf"][4] = (
                    s["rbuf2"][...]
                    + s["rbuf1"][0].astype(jnp.float32)
                ).astype(jnp.bfloat16)
                pl.semaphore_signal(
                    credit2.at[s["idx"]], inc=1, device_id=(s["p2"],),
                    device_id_type=pl.DeviceIdType.MESH)
                pl.semaphore_wait(credit2.at[s["idx"]], 1)
                rdma2 = pltpu.make_async_remote_copy(
                    src_ref=s["sbuf"].at[4],
                    dst_ref=s["rbuf2b"],
                    send_sem=s2_send_sems.at[p, s["idx"]],
                    recv_sem=s2_recv_sems.at[p, s["idx"]],
                    device_id=(s["p2"],),
                    device_id_type=pl.DeviceIdType.MESH,
                )
                rdma2.start()
                s2_prev[s["idx"]] = rdma2
            if p < PH - 1:
                nbase = 2 * ((p + 1) % 2)
                for slot in (0, 1):
                    for s in schemes:
                        s["sbuf"][nbase + slot] = chunk_dot(
                            s["s1_chunks"][slot],
                            colsl(s, p + 1)).astype(jnp.bfloat16)
            for s in schemes:
                s1[s["idx"]][1].wait()
                out_ref[:, colsl(s, p)] = (
                    out_ref[:, colsl(s, p)]
                    + s["rbuf1"][1].astype(jnp.float32))
                if p < PH - 1:
                    pl.semaphore_signal(
                        credit1.at[s["idx"]], inc=1, device_id=(s["p1"],),
                        device_id_type=pl.DeviceIdType.MESH)

        for s in schemes:
            s2_prev[s["idx"]].wait()
            pcol = colsl(s, PH - 1)
            out_ref[:, pcol] = jnp.maximum(
                out_ref[:, pcol] + s["rbuf2b"][...].astype(jnp.float32), 0.0)

        local_amax = jnp.max(out_ref[...])
        amax_out[...] = jnp.full((8, 128), local_amax, jnp.float32)
        amax_rdmas = []
        for j in range(N_DEV - 1):
            tgt = lax.rem(my + j + 1, N_DEV)
            rdma = pltpu.make_async_remote_copy(
                src_ref=amax_out,
                dst_ref=amax_in.at[j],
                send_sem=amax_send_sems.at[j],
                recv_sem=amax_recv_sems.at[j],
                device_id=(tgt,),
                device_id_type=pl.DeviceIdType.MESH,
            )
            rdma.start()
            amax_rdmas.append(rdma)
        for rdma in amax_rdmas:
            rdma.wait()
        gmax = jnp.maximum(local_amax, jnp.max(amax_in[...]))

        scale = gmax / 127.0
        for h in range(2 * PH):
            sl = pl.ds(h * Npc, Npc)
            out_ref[:, sl] = jnp.clip(
                jnp.round(out_ref[:, sl] / scale), -127.0, 127.0) * scale

    return pl.pallas_call(
        body,
        out_shape=jax.ShapeDtypeStruct((Mo, N), jnp.float32),
        in_specs=[
            pl.BlockSpec(memory_space=pltpu.VMEM),
            pl.BlockSpec(memory_space=pltpu.VMEM),
        ],
        out_specs=pl.BlockSpec(memory_space=pltpu.VMEM),
        scratch_shapes=[
            pltpu.VMEM((5, Mo, Npc), jnp.bfloat16),
            pltpu.VMEM((2, Mo, Npc), jnp.bfloat16),
            pltpu.VMEM((Mo, Npc), jnp.float32),
            pltpu.VMEM((Mo, Npc), jnp.bfloat16),
            pltpu.VMEM((5, Mo, Npc), jnp.bfloat16),
            pltpu.VMEM((2, Mo, Npc), jnp.bfloat16),
            pltpu.VMEM((Mo, Npc), jnp.float32),
            pltpu.VMEM((Mo, Npc), jnp.bfloat16),
            pltpu.VMEM((8, 128), jnp.float32),
            pltpu.VMEM((N_DEV - 1, 8, 128), jnp.float32),
            pltpu.SemaphoreType.DMA((PH, 2, 2)),
            pltpu.SemaphoreType.DMA((PH, 2, 2)),
            pltpu.SemaphoreType.DMA((PH, 2)),
            pltpu.SemaphoreType.DMA((PH, 2)),
            pltpu.SemaphoreType.REGULAR((2,)),
            pltpu.SemaphoreType.REGULAR((2,)),
            pltpu.SemaphoreType.DMA((N_DEV - 1,)),
            pltpu.SemaphoreType.DMA((N_DEV - 1,)),
        ],
        compiler_params=pltpu.CompilerParams(
            collective_id=0,
            vmem_limit_bytes=67_010_000,
        ),
    )(x, w_mat)
